# baseline (speedup 1.0000x reference)
import functools

import jax
import jax.numpy as jnp
from jax import lax
from jax.experimental import pallas as pl
from jax.experimental.pallas import tpu as pltpu
from jax.experimental.pallas import tpu_sc as plsc

_H = 256
_W = 256
_F = 32


_NC = 2
_PER_CORE = 48 // _NC


def _sc_gather_body(frames_hbm, out_hbm, buf, sem):
    w = lax.axis_index("c")
    base = w * _PER_CORE
    reads = []
    for i in range(_PER_CORE):
        q = base + i
        c = lax.div(q, 16)
        j = q - 16 * c
        src = c * 64 + lax.div(63 * j, 15)
        reads.append(
            pltpu.async_copy(
                frames_hbm.at[pl.ds(src, 1)], buf.at[pl.ds(i, 1)], sem
            )
        )
    for d in reads:
        d.wait()
    pltpu.sync_copy(buf, out_hbm.at[pl.ds(base, _PER_CORE)])


_sc_gather = functools.partial(
    pl.kernel,
    mesh=plsc.ScalarSubcoreMesh(axis_name="c", num_cores=_NC),
    out_type=jax.ShapeDtypeStruct((48, _H, _W), jnp.float32),
    scratch_types=[
        pltpu.VMEM_SHARED((_PER_CORE, _H, _W), jnp.float32),
        pltpu.SemaphoreType.DMA,
    ],
)(_sc_gather_body)


def _copy_body(in_ref, fast_ref):
    fast_ref[...] = in_ref[...]


def _tc_copy(frames_flat):
    n_blocks = frames_flat.shape[0] // _F
    return pl.pallas_call(
        _copy_body,
        grid=(n_blocks,),
        in_specs=[pl.BlockSpec((_F, _H, _W), lambda k: (k, 0, 0))],
        out_specs=pl.BlockSpec((_F, _H, _W), lambda k: (k, 0, 0)),
        out_shape=jax.ShapeDtypeStruct((n_blocks * _F, _H, _W), jnp.float32),
        compiler_params=pltpu.CompilerParams(
            dimension_semantics=("arbitrary",),
        ),
    )(frames_flat)


def kernel(frames):
    c, t, h, w = frames.shape
    flat = frames.reshape(c * t, h, w)
    slow = _sc_gather(flat)
    fast = _tc_copy(flat)
    return (
        slow.reshape(c, t // 4, h, w),
        fast.reshape(c, t, h, w),
    )
